# baseline (speedup 1.0000x reference)
import functools

import jax
import jax.numpy as jnp
from jax import lax
from jax.experimental import pallas as pl
from jax.experimental.pallas import tpu as pltpu
from jax.experimental.pallas import tpu_sc as plsc

NUM_CLASSES = 1000
N_PROPS = 128
N_IN = N_PROPS + 1
N_OUT = NUM_CLASSES + N_PROPS
BATCH = 16384

NC = 2
NS = 16
L = 16
NW = NC * NS

CHUNK = 64
ROWS_PER_W = BATCH // NW
N_CHUNKS = ROWS_PER_W // CHUNK


def _sc_body(x_hbm, out_hbm, xv, obuf):
    wid = lax.axis_index("s") * NC + lax.axis_index("c")
    iota = lax.iota(jnp.int32, L)
    zeros_f = jnp.zeros((L,), jnp.float32)
    ones_f = jnp.ones((L,), jnp.float32)
    zcols = jnp.zeros((L,), jnp.int32)

    def zrow(r, carry):
        for j in range(62):
            obuf[r, pl.ds(16 * j, L)] = zeros_f
        obuf[r, pl.ds(984, L)] = zeros_f
        return carry

    lax.fori_loop(0, CHUNK, zrow, 0)

    def chunk_body(k, carry):
        base = wid * ROWS_PER_W + k * CHUNK
        pltpu.sync_copy(x_hbm.at[pl.ds(base, CHUNK)], xv)

        def prow(r, c):
            for j in range(8):
                obuf[r, pl.ds(NUM_CLASSES + 16 * j, L)] = xv[r, pl.ds(1 + 16 * j, L)]
            return c

        lax.fori_loop(0, CHUNK, prow, 0)

        for g in range(CHUNK // L):
            rows = iota + g * L
            ids = plsc.load_gather(xv, [rows, zcols]).astype(jnp.int32)
            plsc.store_scatter(obuf, [rows, ids], ones_f)

        pltpu.sync_copy(obuf, out_hbm.at[pl.ds(base, CHUNK)])

        for g in range(CHUNK // L):
            rows = iota + g * L
            ids = plsc.load_gather(xv, [rows, zcols]).astype(jnp.int32)
            plsc.store_scatter(obuf, [rows, ids], zeros_f)
        return carry

    lax.fori_loop(0, N_CHUNKS, chunk_body, 0)


@functools.partial(jax.jit, donate_argnums=())
def _sc_call(X):
    fn = pl.kernel(
        _sc_body,
        out_type=jax.ShapeDtypeStruct((BATCH, N_OUT), jnp.float32),
        mesh=plsc.VectorSubcoreMesh(core_axis_name="c", subcore_axis_name="s"),
        scratch_types=[
            pltpu.VMEM((CHUNK, N_IN), jnp.float32),
            pltpu.VMEM((CHUNK, N_OUT), jnp.float32),
        ],
        compiler_params=pltpu.CompilerParams(
            use_tc_tiling_on_sc=False, needs_layout_passes=False
        ),
    )
    return fn(X)


def kernel(X):
    assert X.shape == (BATCH, N_IN) and X.dtype == jnp.float32
    return _sc_call(X)


# speedup vs baseline: 4.7525x; 4.7525x over previous
import jax
import jax.numpy as jnp
from jax import lax
from jax.experimental import pallas as pl
from jax.experimental.pallas import tpu as pltpu
from jax.experimental.pallas import tpu_sc as plsc

NUM_CLASSES = 1000
N_PROPS = 128
N_IN = N_PROPS + 1
N_OUT = NUM_CLASSES + N_PROPS
BATCH = 16384

NC = 2
NS = 16
L = 16
NW = NC * NS

CHUNK = 128
COLS_PER_W = BATCH // NW
N_CHUNKS = COLS_PER_W // CHUNK
N_SEG = 7
TAIL_OH = NUM_CLASSES - 128 * N_SEG
TAIL_ROWS = TAIL_OH + N_PROPS


def _sc_body(xt_hbm, last_hbm, outT_hbm, xin, lastv, pp, tail, semA, semB, semT):
    wid = lax.axis_index("s") * NC + lax.axis_index("c")
    zeros_f = jnp.zeros((L,), jnp.float32)
    ones_f = jnp.ones((L,), jnp.float32)
    sems = [semA, semB]

    def zrow(r, carry):
        for b in range(8):
            pp[0, r, pl.ds(16 * b, L)] = zeros_f
            pp[1, r, pl.ds(16 * b, L)] = zeros_f
        return carry

    lax.fori_loop(0, CHUNK, zrow, 0)

    def ztail(r, carry):
        for b in range(8):
            tail[r, pl.ds(16 * b, L)] = zeros_f
        return carry

    lax.fori_loop(0, TAIL_OH, ztail, 0)

    def chunk_body(k, carry):
        base = wid * COLS_PER_W + k * CHUNK
        pltpu.sync_copy(xt_hbm.at[pl.ds(0, CHUNK), pl.ds(base, CHUNK)], xin)
        pltpu.sync_copy(last_hbm.at[pl.ds(base, CHUNK)], lastv)

        idvs = []
        for g in range(CHUNK // L):
            cols = lax.iota(jnp.int32, L) + g * L
            ids = xin[0, pl.ds(g * L, L)].astype(jnp.int32)
            idvs.append((cols, lax.shift_right_logical(ids, 7),
                         lax.bitwise_and(ids, 127)))

        def tj(j, c):
            for b in range(8):
                tail[TAIL_OH + j, pl.ds(16 * b, L)] = xin[j + 1, pl.ds(16 * b, L)]
            return c

        lax.fori_loop(0, N_PROPS - 1, tj, 0)
        for b in range(8):
            tail[TAIL_OH + N_PROPS - 1, pl.ds(16 * b, L)] = lastv[pl.ds(16 * b, L)]

        def scat(buf, t, val):
            for cols, hi, lo in idvs:
                plsc.store_scatter(buf, [lo, cols], val, mask=hi == t)

        handles = {}
        for t in range(N_SEG):
            buf = pp.at[t % 2]
            if t >= 2:
                handles[t - 2].wait()
                scat(buf, t - 2, zeros_f)
            scat(buf, t, ones_f)
            h = pltpu.make_async_copy(
                buf, outT_hbm.at[pl.ds(128 * t, 128), pl.ds(base, CHUNK)],
                sems[t % 2])
            h.start()
            handles[t] = h

        scat(tail.at[pl.ds(0, TAIL_OH)], N_SEG, ones_f)
        hT = pltpu.make_async_copy(
            tail, outT_hbm.at[pl.ds(128 * N_SEG, TAIL_ROWS), pl.ds(base, CHUNK)],
            semT)
        hT.start()

        for t in (N_SEG - 2, N_SEG - 1):
            handles[t].wait()
            scat(pp.at[t % 2], t, zeros_f)
        hT.wait()
        scat(tail.at[pl.ds(0, TAIL_OH)], N_SEG, zeros_f)
        return carry

    lax.fori_loop(0, N_CHUNKS, chunk_body, 0)


def _sc_call(XT, last):
    fn = pl.kernel(
        _sc_body,
        out_type=jax.ShapeDtypeStruct((N_OUT, BATCH), jnp.float32),
        mesh=plsc.VectorSubcoreMesh(core_axis_name="c", subcore_axis_name="s"),
        scratch_types=[
            pltpu.VMEM((CHUNK, 128), jnp.float32),
            pltpu.VMEM((CHUNK,), jnp.float32),
            pltpu.VMEM((2, CHUNK, 128), jnp.float32),
            pltpu.VMEM((TAIL_ROWS, 128), jnp.float32),
            pltpu.SemaphoreType.DMA,
            pltpu.SemaphoreType.DMA,
            pltpu.SemaphoreType.DMA,
        ],
        compiler_params=pltpu.CompilerParams(
            use_tc_tiling_on_sc=True, needs_layout_passes=False
        ),
    )
    return fn(XT, last)


@jax.jit
def _run(X):
    outT = _sc_call(X.T, X[:, 128])
    return outT.T


def kernel(X):
    assert X.shape == (BATCH, N_IN) and X.dtype == jnp.float32
    return _run(X)


# speedup vs baseline: 5.5264x; 1.1628x over previous
import jax
import jax.numpy as jnp
from jax import lax
from jax.experimental import pallas as pl
from jax.experimental.pallas import tpu as pltpu
from jax.experimental.pallas import tpu_sc as plsc

NUM_CLASSES = 1000
N_PROPS = 128
N_IN = N_PROPS + 1
N_OUT = NUM_CLASSES + N_PROPS
BATCH = 16384

NC = 2
NS = 16
L = 16
NW = NC * NS

CHUNK = 128
COLS_PER_W = BATCH // NW
N_CHUNKS = COLS_PER_W // CHUNK
N_SEG = 7
TAIL_OH = NUM_CLASSES - 128 * N_SEG
TAIL_ROWS = TAIL_OH + N_PROPS
NG = CHUNK // L


def _sc_body(xt_hbm, last_hbm, outT_hbm, xin, lastv, pp, tail,
             semA, semB, semT, semIA, semIB):
    wid = lax.axis_index("s") * NC + lax.axis_index("c")
    zeros_f = jnp.zeros((L,), jnp.float32)
    ones_f = jnp.ones((L,), jnp.float32)
    zero_ids = [jnp.zeros((L,), jnp.int32)] * (2 * NG)
    sems = [semA, semB]
    isems = [semIA, semIB]
    w0 = wid * COLS_PER_W

    def in_copies(k, slot):
        base = w0 + k * CHUNK
        return (
            pltpu.make_async_copy(
                xt_hbm.at[pl.ds(0, CHUNK), pl.ds(base, CHUNK)],
                xin.at[slot], isems[slot]),
            pltpu.make_async_copy(
                last_hbm.at[pl.ds(base, CHUNK)], lastv.at[slot], isems[slot]),
        )

    def seg_copy(t, base, buf):
        return pltpu.make_async_copy(
            buf, outT_hbm.at[pl.ds(128 * t, 128), pl.ds(base, CHUNK)],
            sems[t % 2])

    def tail_copy(base):
        return pltpu.make_async_copy(
            tail, outT_hbm.at[pl.ds(128 * N_SEG, TAIL_ROWS), pl.ds(base, CHUNK)],
            semT)

    for c in in_copies(0, 0):
        c.start()
    for c in in_copies(1, 1):
        c.start()

    def zrow(r, carry):
        for b in range(NG):
            pp[0, r, pl.ds(16 * b, L)] = zeros_f
            pp[1, r, pl.ds(16 * b, L)] = zeros_f
        return carry

    lax.fori_loop(0, CHUNK, zrow, 0)

    def ztail(r, carry):
        for b in range(NG):
            tail[r, pl.ds(16 * b, L)] = zeros_f
        return carry

    lax.fori_loop(0, TAIL_OH, ztail, 0)

    def scat(buf, ids, t, val):
        for g in range(NG):
            cols = lax.iota(jnp.int32, L) + g * L
            plsc.store_scatter(buf, [ids[NG + g], cols], val,
                               mask=ids[g] == t)

    def chunk(k, slot, prev_ids, first, xbuf, lbuf):
        base = w0 + k * CHUNK
        for c in in_copies(k, slot):
            c.wait()

        his, los = [], []
        for g in range(NG):
            ids = xbuf[0, pl.ds(g * L, L)].astype(jnp.int32)
            his.append(lax.shift_right_logical(ids, 7))
            los.append(lax.bitwise_and(ids, 127))
        ids_k = his + los

        def drain_prev():
            seg_copy(N_SEG - 1, base - CHUNK, pp.at[0]).wait()
            scat(pp.at[0], prev_ids, N_SEG - 1, zeros_f)
            seg_copy(N_SEG - 2, base - CHUNK, pp.at[1]).wait()
            scat(pp.at[1], prev_ids, N_SEG - 2, zeros_f)
            tail_copy(base - CHUNK).wait()
            scat(tail.at[pl.ds(0, TAIL_OH)], prev_ids, N_SEG, zeros_f)

        if first:
            pl.when(k > 0)(drain_prev)
        else:
            drain_prev()

        handles = {}
        for t in range(N_SEG):
            buf = pp.at[t % 2]
            if t >= 2:
                handles[t - 2].wait()
                scat(buf, ids_k, t - 2, zeros_f)
            scat(buf, ids_k, t, ones_f)
            h = seg_copy(t, base, buf)
            h.start()
            handles[t] = h
            if t == 1:
                def tj(j, c):
                    for b in range(NG):
                        tail[TAIL_OH + j, pl.ds(16 * b, L)] = \
                            xbuf[j + 1, pl.ds(16 * b, L)]
                    return c

                lax.fori_loop(0, N_PROPS - 1, tj, 0)
                for b in range(NG):
                    tail[TAIL_OH + N_PROPS - 1, pl.ds(16 * b, L)] = \
                        lbuf[pl.ds(16 * b, L)]
                def prefetch():
                    for c in in_copies(k + 2, slot):
                        c.start()

                pl.when(k + 2 < N_CHUNKS)(prefetch)

        scat(tail.at[pl.ds(0, TAIL_OH)], ids_k, N_SEG, ones_f)
        tail_copy(base).start()
        return ids_k

    def pair(i, carry):
        ids_a = chunk(2 * i, 0, list(carry), True, xin.at[0], lastv.at[0])
        ids_b = chunk(2 * i + 1, 1, ids_a, False, xin.at[1], lastv.at[1])
        return tuple(ids_b)

    final_ids = lax.fori_loop(0, N_CHUNKS // 2, pair, tuple(zero_ids))

    last_base = w0 + (N_CHUNKS - 1) * CHUNK
    seg_copy(N_SEG - 1, last_base, pp.at[0]).wait()
    seg_copy(N_SEG - 2, last_base, pp.at[1]).wait()
    tail_copy(last_base).wait()
    del final_ids


def _sc_call(XT, last):
    fn = pl.kernel(
        _sc_body,
        out_type=jax.ShapeDtypeStruct((N_OUT, BATCH), jnp.float32),
        mesh=plsc.VectorSubcoreMesh(core_axis_name="c", subcore_axis_name="s"),
        scratch_types=[
            pltpu.VMEM((2, CHUNK, 128), jnp.float32),
            pltpu.VMEM((2, CHUNK), jnp.float32),
            pltpu.VMEM((2, CHUNK, 128), jnp.float32),
            pltpu.VMEM((TAIL_ROWS, 128), jnp.float32),
            pltpu.SemaphoreType.DMA,
            pltpu.SemaphoreType.DMA,
            pltpu.SemaphoreType.DMA,
            pltpu.SemaphoreType.DMA,
            pltpu.SemaphoreType.DMA,
        ],
        compiler_params=pltpu.CompilerParams(
            use_tc_tiling_on_sc=True, needs_layout_passes=False
        ),
    )
    return fn(XT, last)


@jax.jit
def _run(X):
    outT = _sc_call(X.T, X[:, 128])
    return outT.T


def kernel(X):
    assert X.shape == (BATCH, N_IN) and X.dtype == jnp.float32
    return _run(X)
